# baseline (speedup 1.0000x reference)
import jax
import jax.numpy as jnp
from jax import lax
from jax.experimental import pallas as pl
from jax.experimental.pallas import tpu as pltpu

_EPS = 1e-8
_B = 4096
_INF = 1e30
_INTERPRET = False


def _roll4(c):
    return jnp.concatenate([c[1:4], c[0:1]], axis=0)


def _sincos(a):
    q = jnp.round(a * (2.0 / jnp.pi))
    k = q.astype(jnp.int32) & 3
    r = a - q * 1.5707963705062866
    r = r + q * 4.3711388286737929e-08
    x2 = r * r
    sp = r + r * x2 * (-1.6666654611e-1 + x2 * (8.3321608736e-3
                                                + x2 * -1.9515295891e-4))
    cp = 1.0 + x2 * (-0.5 + x2 * (4.166664568298827e-2
                                  + x2 * (-1.388731625493765e-3
                                          + x2 * 2.443315711809948e-5)))
    k1 = k == 1
    k2 = k == 2
    k3 = k == 3
    s = jnp.where(k1, cp, jnp.where(k2, -sp, jnp.where(k3, -cp, sp)))
    c = jnp.where(k1, -sp, jnp.where(k2, -cp, jnp.where(k3, sp, cp)))
    return s, c


def _corners(x, y, w, l, a):
    s, c = _sincos(a)
    hw = 0.5 * w
    hl = 0.5 * l
    xs = jnp.stack([hw, -hw, -hw, hw], axis=0)
    ys = jnp.stack([hl, hl, -hl, -hl], axis=0)
    cx = x + xs * c - ys * s
    cy = y + xs * s + ys * c
    return cx, cy


def _clip_contrib(pxc, pyc, qxc, qyc):
    q2x, q2y = _roll4(qxc), _roll4(qyc)
    ex, ey = q2x - qxc, q2y - qyc
    cj = ey * qxc - ex * qyc
    t0 = t1 = None
    for j in range(4):
        s_a = ey[j] * pxc - ex[j] * pyc - cj[j]
        s_b = _roll4(s_a)
        d = s_b - s_a
        dzer = jnp.abs(d) < 1e-30
        inv = 1.0 / jnp.where(dzer, 1.0, d)
        tc = -s_a * inv
        upper = jnp.where(d > 1e-30, tc,
                          jnp.where(dzer & (s_a > 0), -_INF, _INF))
        lower = jnp.where(d < -1e-30, tc, -_INF)
        t1 = upper if t1 is None else jnp.minimum(t1, upper)
        t0 = lower if t0 is None else jnp.maximum(t0, lower)
    t0f = jnp.clip(t0, 0.0, 1.0)
    t1f = jnp.maximum(jnp.clip(t1, 0.0, 1.0), t0f)
    p2x, p2y = _roll4(pxc), _roll4(pyc)
    dt = t1f - t0f
    qd = 0.5 * (t1f * t1f - t0f * t0f)
    contrib = (p2y - pyc) * (pxc * dt + (p2x - pxc) * qd)
    return jnp.sum(contrib, axis=0)


def _body(data_ref, out_ref):
    f = data_ref[...]
    x1, y1, z1, w1, l1, h1, a1 = (f[i] for i in range(7))
    x2, y2, z2, w2, l2, h2, a2 = (f[i] for i in range(7, 14))
    wt = f[14]

    cx1, cy1 = _corners(x1, y1, w1, l1, a1)
    cx2, cy2 = _corners(x2, y2, w2, l2, a2)

    area = _clip_contrib(cx1, cy1, cx2, cy2) + _clip_contrib(cx2, cy2, cx1, cy1)
    area = jnp.maximum(area, 0.0)

    zt = jnp.minimum(z1 + 0.5 * h1, z2 + 0.5 * h2)
    zb = jnp.maximum(z1 - 0.5 * h1, z2 - 0.5 * h2)
    vol = area * jnp.maximum(zt - zb, 0.0)
    v1 = w1 * l1 * h1
    v2 = w2 * l2 * h2
    iou = vol / (v1 + v2 - vol + _EPS)
    loss = (1.0 - iou) * wt

    s = jnp.sum(loss)
    lane = lax.broadcasted_iota(jnp.int32, (1, 1, 128), 2)
    out_ref[...] = jnp.where(lane == 0, s, 0.0)


def kernel(pred, target, weight):
    n = pred.shape[0]
    b = _B
    g2 = -(-n // (2 * b))
    nblk = 2 * g2
    npad = nblk * b
    padn = npad - n

    padp = jnp.array([0.0, 0.0, 0.0, 1.0, 1.0, 1.0, 0.0], jnp.float32)
    padt = jnp.array([10.0, 10.0, 10.0, 1.0, 1.0, 1.0, 0.0], jnp.float32)
    pr = pred.astype(jnp.float32)
    tg = target.astype(jnp.float32)
    wt = weight.astype(jnp.float32)
    if padn:
        pr = jnp.concatenate([pr, jnp.broadcast_to(padp, (padn, 7))], axis=0)
        tg = jnp.concatenate([tg, jnp.broadcast_to(padt, (padn, 7))], axis=0)
        wt = jnp.concatenate([wt, jnp.zeros((padn,), jnp.float32)], axis=0)
    data = jnp.concatenate(
        [pr.T, tg.T, wt[None, :], jnp.zeros((1, npad), jnp.float32)], axis=0)
    data = data.reshape(16, npad // 128, 128)

    rows = b // 128
    out = pl.pallas_call(
        _body,
        grid=(2, g2),
        in_specs=[
            pl.BlockSpec((16, rows, 128), lambda c, g: (0, c * g2 + g, 0)),
        ],
        out_specs=pl.BlockSpec((1, 1, 128), lambda c, g: (c * g2 + g, 0, 0)),
        out_shape=jax.ShapeDtypeStruct((nblk, 1, 128), jnp.float32),
        compiler_params=pltpu.CompilerParams(
            dimension_semantics=("parallel", "arbitrary")),
        interpret=_INTERPRET,
    )(data)
    return jnp.sum(out) / jnp.float32(n)


# speedup vs baseline: 1.2342x; 1.2342x over previous
import jax
import jax.numpy as jnp
from jax import lax
from jax.experimental import pallas as pl
from jax.experimental.pallas import tpu as pltpu

_EPS = 1e-8
_B = 8192
_INF = 1e30
_INTERPRET = False


def _roll4(c):
    return jnp.concatenate([c[1:4], c[0:1]], axis=0)


def _sincos(a):
    q = jnp.round(a * (2.0 / jnp.pi))
    k = q.astype(jnp.int32) & 3
    r = a - q * 1.5707963705062866
    r = r + q * 4.3711388286737929e-08
    x2 = r * r
    sp = r + r * x2 * (-1.6666654611e-1 + x2 * (8.3321608736e-3
                                                + x2 * -1.9515295891e-4))
    cp = 1.0 + x2 * (-0.5 + x2 * (4.166664568298827e-2
                                  + x2 * (-1.388731625493765e-3
                                          + x2 * 2.443315711809948e-5)))
    k1 = k == 1
    k2 = k == 2
    k3 = k == 3
    s = jnp.where(k1, cp, jnp.where(k2, -sp, jnp.where(k3, -cp, sp)))
    c = jnp.where(k1, -sp, jnp.where(k2, -cp, jnp.where(k3, sp, cp)))
    return s, c


def _corners(x, y, w, l, a):
    s, c = _sincos(a)
    hw = 0.5 * w
    hl = 0.5 * l
    xs = jnp.stack([hw, -hw, -hw, hw], axis=0)
    ys = jnp.stack([hl, hl, -hl, -hl], axis=0)
    cx = x + xs * c - ys * s
    cy = y + xs * s + ys * c
    return cx, cy


def _clip_contrib(pxc, pyc, qxc, qyc):
    q2x, q2y = _roll4(qxc), _roll4(qyc)
    ex, ey = q2x - qxc, q2y - qyc
    cj = ey * qxc - ex * qyc
    t0 = t1 = None
    for j in range(4):
        s_a = ey[j] * pxc - ex[j] * pyc - cj[j]
        s_b = _roll4(s_a)
        d = s_b - s_a
        dzer = jnp.abs(d) < 1e-30
        inv = 1.0 / jnp.where(dzer, 1.0, d)
        tc = -s_a * inv
        upper = jnp.where(d > 1e-30, tc,
                          jnp.where(dzer & (s_a > 0), -_INF, _INF))
        lower = jnp.where(d < -1e-30, tc, -_INF)
        t1 = upper if t1 is None else jnp.minimum(t1, upper)
        t0 = lower if t0 is None else jnp.maximum(t0, lower)
    t0f = jnp.clip(t0, 0.0, 1.0)
    t1f = jnp.maximum(jnp.clip(t1, 0.0, 1.0), t0f)
    p2x, p2y = _roll4(pxc), _roll4(pyc)
    dt = t1f - t0f
    qd = 0.5 * (t1f * t1f - t0f * t0f)
    contrib = (p2y - pyc) * (pxc * dt + (p2x - pxc) * qd)
    return jnp.sum(contrib, axis=0)


def _loss_tile(f):
    x1, y1, z1, w1, l1, h1, a1 = (f[i] for i in range(7))
    x2, y2, z2, w2, l2, h2, a2 = (f[i] for i in range(7, 14))
    wt = f[14]

    cx1, cy1 = _corners(x1, y1, w1, l1, a1)
    cx2, cy2 = _corners(x2, y2, w2, l2, a2)

    area = _clip_contrib(cx1, cy1, cx2, cy2) + _clip_contrib(cx2, cy2, cx1, cy1)
    area = jnp.maximum(area, 0.0)

    zt = jnp.minimum(z1 + 0.5 * h1, z2 + 0.5 * h2)
    zb = jnp.maximum(z1 - 0.5 * h1, z2 - 0.5 * h2)
    vol = area * jnp.maximum(zt - zb, 0.0)
    v1 = w1 * l1 * h1
    v2 = w2 * l2 * h2
    iou = vol / (v1 + v2 - vol + _EPS)
    return (1.0 - iou) * wt


_CHUNK = 16


def _body(data_ref, out_ref):
    f = data_ref[...]
    rows = f.shape[1]
    acc = None
    for c0 in range(0, rows, _CHUNK):
        part = jnp.sum(_loss_tile(f[:, c0:c0 + _CHUNK, :]), axis=0,
                       keepdims=True)
        acc = part if acc is None else acc + part
    s = jnp.sum(acc)
    lane = lax.broadcasted_iota(jnp.int32, (1, 1, 128), 2)
    out_ref[...] = jnp.where(lane == 0, s, 0.0)


def kernel(pred, target, weight):
    n = pred.shape[0]
    b = _B
    g2 = -(-n // (2 * b))
    nblk = 2 * g2
    npad = nblk * b
    padn = npad - n

    padp = jnp.array([0.0, 0.0, 0.0, 1.0, 1.0, 1.0, 0.0], jnp.float32)
    padt = jnp.array([10.0, 10.0, 10.0, 1.0, 1.0, 1.0, 0.0], jnp.float32)
    pr = pred.astype(jnp.float32)
    tg = target.astype(jnp.float32)
    wt = weight.astype(jnp.float32)
    if padn:
        pr = jnp.concatenate([pr, jnp.broadcast_to(padp, (padn, 7))], axis=0)
        tg = jnp.concatenate([tg, jnp.broadcast_to(padt, (padn, 7))], axis=0)
        wt = jnp.concatenate([wt, jnp.zeros((padn,), jnp.float32)], axis=0)
    data = jnp.concatenate(
        [pr.T, tg.T, wt[None, :], jnp.zeros((1, npad), jnp.float32)], axis=0)
    data = data.reshape(16, npad // 128, 128)

    rows = b // 128
    out = pl.pallas_call(
        _body,
        grid=(2, g2),
        in_specs=[
            pl.BlockSpec((16, rows, 128), lambda c, g: (0, c * g2 + g, 0)),
        ],
        out_specs=pl.BlockSpec((1, 1, 128), lambda c, g: (c * g2 + g, 0, 0)),
        out_shape=jax.ShapeDtypeStruct((nblk, 1, 128), jnp.float32),
        compiler_params=pltpu.CompilerParams(
            dimension_semantics=("parallel", "arbitrary")),
        interpret=_INTERPRET,
    )(data)
    return jnp.sum(out) / jnp.float32(n)


# speedup vs baseline: 1.2445x; 1.0083x over previous
import jax
import jax.numpy as jnp
from jax import lax
from jax.experimental import pallas as pl
from jax.experimental.pallas import tpu as pltpu

_EPS = 1e-8
_B = 8192
_INF = 1e30
_INTERPRET = False


def _roll4(c):
    return jnp.concatenate([c[1:4], c[0:1]], axis=0)


def _sincos(a):
    q = jnp.round(a * (2.0 / jnp.pi))
    k = q.astype(jnp.int32) & 3
    r = a - q * 1.5707963705062866
    r = r + q * 4.3711388286737929e-08
    x2 = r * r
    sp = r + r * x2 * (-1.6666654611e-1 + x2 * (8.3321608736e-3
                                                + x2 * -1.9515295891e-4))
    cp = 1.0 + x2 * (-0.5 + x2 * (4.166664568298827e-2
                                  + x2 * (-1.388731625493765e-3
                                          + x2 * 2.443315711809948e-5)))
    k1 = k == 1
    k2 = k == 2
    k3 = k == 3
    s = jnp.where(k1, cp, jnp.where(k2, -sp, jnp.where(k3, -cp, sp)))
    c = jnp.where(k1, -sp, jnp.where(k2, -cp, jnp.where(k3, sp, cp)))
    return s, c


def _corners(x, y, w, l, a):
    s, c = _sincos(a)
    hw = 0.5 * w
    hl = 0.5 * l
    xs = jnp.stack([hw, -hw, -hw, hw], axis=0)
    ys = jnp.stack([hl, hl, -hl, -hl], axis=0)
    cx = x + xs * c - ys * s
    cy = y + xs * s + ys * c
    return cx, cy


def _clip_contrib(pxc, pyc, qxc, qyc):
    q2x, q2y = _roll4(qxc), _roll4(qyc)
    ex, ey = q2x - qxc, q2y - qyc
    cj = ey * qxc - ex * qyc
    t0 = t1 = None
    for j in range(4):
        s_a = ey[j] * pxc - ex[j] * pyc - cj[j]
        s_b = _roll4(s_a)
        d = s_b - s_a
        dzer = jnp.abs(d) < 1e-30
        inv = 1.0 / jnp.where(dzer, 1.0, d)
        tc = -s_a * inv
        upper = jnp.where(d > 1e-30, tc,
                          jnp.where(dzer & (s_a > 0), -_INF, _INF))
        lower = jnp.where(d < -1e-30, tc, -_INF)
        t1 = upper if t1 is None else jnp.minimum(t1, upper)
        t0 = lower if t0 is None else jnp.maximum(t0, lower)
    t0f = jnp.clip(t0, 0.0, 1.0)
    t1f = jnp.maximum(jnp.clip(t1, 0.0, 1.0), t0f)
    p2x, p2y = _roll4(pxc), _roll4(pyc)
    dt = t1f - t0f
    qd = 0.5 * (t1f * t1f - t0f * t0f)
    contrib = (p2y - pyc) * (pxc * dt + (p2x - pxc) * qd)
    return jnp.sum(contrib, axis=0)


def _loss_tile(f):
    x1, y1, z1, w1, l1, h1, a1 = (f[i] for i in range(7))
    x2, y2, z2, w2, l2, h2, a2 = (f[i] for i in range(7, 14))
    wt = f[14]

    cx1, cy1 = _corners(x1, y1, w1, l1, a1)
    cx2, cy2 = _corners(x2, y2, w2, l2, a2)

    area = _clip_contrib(cx1, cy1, cx2, cy2) + _clip_contrib(cx2, cy2, cx1, cy1)
    area = jnp.maximum(area, 0.0)

    zt = jnp.minimum(z1 + 0.5 * h1, z2 + 0.5 * h2)
    zb = jnp.maximum(z1 - 0.5 * h1, z2 - 0.5 * h2)
    vol = area * jnp.maximum(zt - zb, 0.0)
    v1 = w1 * l1 * h1
    v2 = w2 * l2 * h2
    iou = vol / (v1 + v2 - vol + _EPS)
    return (1.0 - iou) * wt


_CHUNK = 8


def _body(data_ref, out_ref):
    f = data_ref[...]
    rows = f.shape[1]
    acc = None
    for c0 in range(0, rows, _CHUNK):
        part = jnp.sum(_loss_tile(f[:, c0:c0 + _CHUNK, :]), axis=0,
                       keepdims=True)
        acc = part if acc is None else acc + part
    s = jnp.sum(acc)
    lane = lax.broadcasted_iota(jnp.int32, (1, 1, 128), 2)
    out_ref[...] = jnp.where(lane == 0, s, 0.0)


def kernel(pred, target, weight):
    n = pred.shape[0]
    b = _B
    g2 = -(-n // (2 * b))
    nblk = 2 * g2
    npad = nblk * b
    padn = npad - n

    padp = jnp.array([0.0, 0.0, 0.0, 1.0, 1.0, 1.0, 0.0], jnp.float32)
    padt = jnp.array([10.0, 10.0, 10.0, 1.0, 1.0, 1.0, 0.0], jnp.float32)
    pr = pred.astype(jnp.float32)
    tg = target.astype(jnp.float32)
    wt = weight.astype(jnp.float32)
    if padn:
        pr = jnp.concatenate([pr, jnp.broadcast_to(padp, (padn, 7))], axis=0)
        tg = jnp.concatenate([tg, jnp.broadcast_to(padt, (padn, 7))], axis=0)
        wt = jnp.concatenate([wt, jnp.zeros((padn,), jnp.float32)], axis=0)
    data = jnp.concatenate(
        [pr.T, tg.T, wt[None, :], jnp.zeros((1, npad), jnp.float32)], axis=0)
    data = data.reshape(16, npad // 128, 128)

    rows = b // 128
    out = pl.pallas_call(
        _body,
        grid=(2, g2),
        in_specs=[
            pl.BlockSpec((16, rows, 128), lambda c, g: (0, c * g2 + g, 0)),
        ],
        out_specs=pl.BlockSpec((1, 1, 128), lambda c, g: (c * g2 + g, 0, 0)),
        out_shape=jax.ShapeDtypeStruct((nblk, 1, 128), jnp.float32),
        compiler_params=pltpu.CompilerParams(
            dimension_semantics=("parallel", "arbitrary")),
        interpret=_INTERPRET,
    )(data)
    return jnp.sum(out) / jnp.float32(n)


# speedup vs baseline: 1.2596x; 1.0121x over previous
import jax
import jax.numpy as jnp
from jax import lax
from jax.experimental import pallas as pl
from jax.experimental.pallas import tpu as pltpu

_EPS = 1e-8
_B = 16384
_INF = 1e30
_INTERPRET = False


def _roll4(c):
    return jnp.concatenate([c[1:4], c[0:1]], axis=0)


def _sincos(a):
    q = jnp.round(a * (2.0 / jnp.pi))
    k = q.astype(jnp.int32) & 3
    r = a - q * 1.5707963705062866
    r = r + q * 4.3711388286737929e-08
    x2 = r * r
    sp = r + r * x2 * (-1.6666654611e-1 + x2 * (8.3321608736e-3
                                                + x2 * -1.9515295891e-4))
    cp = 1.0 + x2 * (-0.5 + x2 * (4.166664568298827e-2
                                  + x2 * (-1.388731625493765e-3
                                          + x2 * 2.443315711809948e-5)))
    k1 = k == 1
    k2 = k == 2
    k3 = k == 3
    s = jnp.where(k1, cp, jnp.where(k2, -sp, jnp.where(k3, -cp, sp)))
    c = jnp.where(k1, -sp, jnp.where(k2, -cp, jnp.where(k3, sp, cp)))
    return s, c


def _corners(x, y, w, l, a):
    s, c = _sincos(a)
    hw = 0.5 * w
    hl = 0.5 * l
    xs = jnp.stack([hw, -hw, -hw, hw], axis=0)
    ys = jnp.stack([hl, hl, -hl, -hl], axis=0)
    cx = x + xs * c - ys * s
    cy = y + xs * s + ys * c
    return cx, cy


def _clip_contrib(pxc, pyc, qxc, qyc):
    q2x, q2y = _roll4(qxc), _roll4(qyc)
    ex, ey = q2x - qxc, q2y - qyc
    cj = ey * qxc - ex * qyc
    t0 = t1 = None
    for j in range(4):
        s_a = ey[j] * pxc - ex[j] * pyc - cj[j]
        s_b = _roll4(s_a)
        d = s_b - s_a
        dzer = jnp.abs(d) < 1e-30
        inv = 1.0 / jnp.where(dzer, 1.0, d)
        tc = -s_a * inv
        upper = jnp.where(d > 1e-30, tc,
                          jnp.where(dzer & (s_a > 0), -_INF, _INF))
        lower = jnp.where(d < -1e-30, tc, -_INF)
        t1 = upper if t1 is None else jnp.minimum(t1, upper)
        t0 = lower if t0 is None else jnp.maximum(t0, lower)
    t0f = jnp.clip(t0, 0.0, 1.0)
    t1f = jnp.maximum(jnp.clip(t1, 0.0, 1.0), t0f)
    p2x, p2y = _roll4(pxc), _roll4(pyc)
    dt = t1f - t0f
    qd = 0.5 * (t1f * t1f - t0f * t0f)
    contrib = (p2y - pyc) * (pxc * dt + (p2x - pxc) * qd)
    return jnp.sum(contrib, axis=0)


def _loss_tile(f):
    x1, y1, z1, w1, l1, h1, a1 = (f[i] for i in range(7))
    x2, y2, z2, w2, l2, h2, a2 = (f[i] for i in range(7, 14))
    wt = f[14]

    cx1, cy1 = _corners(x1, y1, w1, l1, a1)
    cx2, cy2 = _corners(x2, y2, w2, l2, a2)

    area = _clip_contrib(cx1, cy1, cx2, cy2) + _clip_contrib(cx2, cy2, cx1, cy1)
    area = jnp.maximum(area, 0.0)

    zt = jnp.minimum(z1 + 0.5 * h1, z2 + 0.5 * h2)
    zb = jnp.maximum(z1 - 0.5 * h1, z2 - 0.5 * h2)
    vol = area * jnp.maximum(zt - zb, 0.0)
    v1 = w1 * l1 * h1
    v2 = w2 * l2 * h2
    iou = vol / (v1 + v2 - vol + _EPS)
    return (1.0 - iou) * wt


_CHUNK = 8


def _body(data_ref, out_ref):
    f = data_ref[...]
    rows = f.shape[1]
    acc = None
    for c0 in range(0, rows, _CHUNK):
        part = jnp.sum(_loss_tile(f[:, c0:c0 + _CHUNK, :]), axis=0,
                       keepdims=True)
        acc = part if acc is None else acc + part
    s = jnp.sum(acc)
    lane = lax.broadcasted_iota(jnp.int32, (1, 1, 128), 2)
    out_ref[...] = jnp.where(lane == 0, s, 0.0)


def kernel(pred, target, weight):
    n = pred.shape[0]
    b = _B
    g2 = -(-n // (2 * b))
    nblk = 2 * g2
    npad = nblk * b
    padn = npad - n

    padp = jnp.array([0.0, 0.0, 0.0, 1.0, 1.0, 1.0, 0.0], jnp.float32)
    padt = jnp.array([10.0, 10.0, 10.0, 1.0, 1.0, 1.0, 0.0], jnp.float32)
    pr = pred.astype(jnp.float32)
    tg = target.astype(jnp.float32)
    wt = weight.astype(jnp.float32)
    if padn:
        pr = jnp.concatenate([pr, jnp.broadcast_to(padp, (padn, 7))], axis=0)
        tg = jnp.concatenate([tg, jnp.broadcast_to(padt, (padn, 7))], axis=0)
        wt = jnp.concatenate([wt, jnp.zeros((padn,), jnp.float32)], axis=0)
    data = jnp.concatenate(
        [pr.T, tg.T, wt[None, :], jnp.zeros((1, npad), jnp.float32)], axis=0)
    data = data.reshape(16, npad // 128, 128)

    rows = b // 128
    out = pl.pallas_call(
        _body,
        grid=(2, g2),
        in_specs=[
            pl.BlockSpec((16, rows, 128), lambda c, g: (0, c * g2 + g, 0)),
        ],
        out_specs=pl.BlockSpec((1, 1, 128), lambda c, g: (c * g2 + g, 0, 0)),
        out_shape=jax.ShapeDtypeStruct((nblk, 1, 128), jnp.float32),
        compiler_params=pltpu.CompilerParams(
            dimension_semantics=("parallel", "arbitrary")),
        interpret=_INTERPRET,
    )(data)
    return jnp.sum(out) / jnp.float32(n)


# speedup vs baseline: 1.2872x; 1.0219x over previous
import jax
import jax.numpy as jnp
from jax import lax
from jax.experimental import pallas as pl
from jax.experimental.pallas import tpu as pltpu

_EPS = 1e-8
_B = 32768
_INF = 1e30
_INTERPRET = False


def _roll4(c):
    return jnp.concatenate([c[1:4], c[0:1]], axis=0)


def _sincos(a):
    q = jnp.round(a * (2.0 / jnp.pi))
    k = q.astype(jnp.int32) & 3
    r = a - q * 1.5707963705062866
    r = r + q * 4.3711388286737929e-08
    x2 = r * r
    sp = r + r * x2 * (-1.6666654611e-1 + x2 * (8.3321608736e-3
                                                + x2 * -1.9515295891e-4))
    cp = 1.0 + x2 * (-0.5 + x2 * (4.166664568298827e-2
                                  + x2 * (-1.388731625493765e-3
                                          + x2 * 2.443315711809948e-5)))
    k1 = k == 1
    k2 = k == 2
    k3 = k == 3
    s = jnp.where(k1, cp, jnp.where(k2, -sp, jnp.where(k3, -cp, sp)))
    c = jnp.where(k1, -sp, jnp.where(k2, -cp, jnp.where(k3, sp, cp)))
    return s, c


def _corners(x, y, w, l, a):
    s, c = _sincos(a)
    hw = 0.5 * w
    hl = 0.5 * l
    xs = jnp.stack([hw, -hw, -hw, hw], axis=0)
    ys = jnp.stack([hl, hl, -hl, -hl], axis=0)
    cx = x + xs * c - ys * s
    cy = y + xs * s + ys * c
    return cx, cy


def _clip_contrib(pxc, pyc, qxc, qyc):
    q2x, q2y = _roll4(qxc), _roll4(qyc)
    ex, ey = q2x - qxc, q2y - qyc
    cj = ey * qxc - ex * qyc
    t0 = t1 = None
    for j in range(4):
        s_a = ey[j] * pxc - ex[j] * pyc - cj[j]
        s_b = _roll4(s_a)
        d = s_b - s_a
        dzer = jnp.abs(d) < 1e-30
        inv = 1.0 / jnp.where(dzer, 1.0, d)
        tc = -s_a * inv
        upper = jnp.where(d > 1e-30, tc,
                          jnp.where(dzer & (s_a > 0), -_INF, _INF))
        lower = jnp.where(d < -1e-30, tc, -_INF)
        t1 = upper if t1 is None else jnp.minimum(t1, upper)
        t0 = lower if t0 is None else jnp.maximum(t0, lower)
    t0f = jnp.clip(t0, 0.0, 1.0)
    t1f = jnp.maximum(jnp.clip(t1, 0.0, 1.0), t0f)
    p2x, p2y = _roll4(pxc), _roll4(pyc)
    dt = t1f - t0f
    qd = 0.5 * (t1f * t1f - t0f * t0f)
    contrib = (p2y - pyc) * (pxc * dt + (p2x - pxc) * qd)
    return jnp.sum(contrib, axis=0)


def _loss_tile(f):
    x1, y1, z1, w1, l1, h1, a1 = (f[i] for i in range(7))
    x2, y2, z2, w2, l2, h2, a2 = (f[i] for i in range(7, 14))
    wt = f[14]

    cx1, cy1 = _corners(x1, y1, w1, l1, a1)
    cx2, cy2 = _corners(x2, y2, w2, l2, a2)

    area = _clip_contrib(cx1, cy1, cx2, cy2) + _clip_contrib(cx2, cy2, cx1, cy1)
    area = jnp.maximum(area, 0.0)

    zt = jnp.minimum(z1 + 0.5 * h1, z2 + 0.5 * h2)
    zb = jnp.maximum(z1 - 0.5 * h1, z2 - 0.5 * h2)
    vol = area * jnp.maximum(zt - zb, 0.0)
    v1 = w1 * l1 * h1
    v2 = w2 * l2 * h2
    iou = vol / (v1 + v2 - vol + _EPS)
    return (1.0 - iou) * wt


_CHUNK = 8


def _body(data_ref, out_ref):
    f = data_ref[...]
    rows = f.shape[1]
    acc = None
    for c0 in range(0, rows, _CHUNK):
        part = jnp.sum(_loss_tile(f[:, c0:c0 + _CHUNK, :]), axis=0,
                       keepdims=True)
        acc = part if acc is None else acc + part
    s = jnp.sum(acc)
    lane = lax.broadcasted_iota(jnp.int32, (1, 1, 128), 2)
    out_ref[...] = jnp.where(lane == 0, s, 0.0)


def kernel(pred, target, weight):
    n = pred.shape[0]
    b = _B
    g2 = -(-n // (2 * b))
    nblk = 2 * g2
    npad = nblk * b
    padn = npad - n

    padp = jnp.array([0.0, 0.0, 0.0, 1.0, 1.0, 1.0, 0.0], jnp.float32)
    padt = jnp.array([10.0, 10.0, 10.0, 1.0, 1.0, 1.0, 0.0], jnp.float32)
    pr = pred.astype(jnp.float32)
    tg = target.astype(jnp.float32)
    wt = weight.astype(jnp.float32)
    if padn:
        pr = jnp.concatenate([pr, jnp.broadcast_to(padp, (padn, 7))], axis=0)
        tg = jnp.concatenate([tg, jnp.broadcast_to(padt, (padn, 7))], axis=0)
        wt = jnp.concatenate([wt, jnp.zeros((padn,), jnp.float32)], axis=0)
    data = jnp.concatenate(
        [pr.T, tg.T, wt[None, :], jnp.zeros((1, npad), jnp.float32)], axis=0)
    data = data.reshape(16, npad // 128, 128)

    rows = b // 128
    out = pl.pallas_call(
        _body,
        grid=(2, g2),
        in_specs=[
            pl.BlockSpec((16, rows, 128), lambda c, g: (0, c * g2 + g, 0)),
        ],
        out_specs=pl.BlockSpec((1, 1, 128), lambda c, g: (c * g2 + g, 0, 0)),
        out_shape=jax.ShapeDtypeStruct((nblk, 1, 128), jnp.float32),
        compiler_params=pltpu.CompilerParams(
            dimension_semantics=("parallel", "arbitrary")),
        interpret=_INTERPRET,
    )(data)
    return jnp.sum(out) / jnp.float32(n)
